# Initial kernel scaffold; baseline (speedup 1.0000x reference)
import jax
import jax.numpy as jnp
from jax.experimental import pallas as pl


def kernel(feat, edge_index, W, b):
    raise NotImplementedError("write your pallas kernel here")


# speedup vs baseline: 2.1326x; 2.1326x over previous
import functools

import jax
import jax.numpy as jnp
from jax import lax
from jax.experimental import pallas as pl
from jax.experimental.pallas import tpu as pltpu
from jax.experimental.pallas import tpu_sc as plsc

_N = 10000
_E = 320000
_D = 128
_K = 16
_ALPHA = 0.05

_NC = 2
_NS = 16
_NW = _NC * _NS
_CHUNK = 128
_CHUNKS_PER_TILE = 80
_EPT = _CHUNK * _CHUNKS_PER_TILE
_EPAD = _EPT * _NW
_PROWS = 10240
_SCRATCH_ROW = _N
_ZROWS = 64
_ROWS_PER_TILE = _PROWS // _NS

_mesh = plsc.VectorSubcoreMesh(core_axis_name="c", subcore_axis_name="s")


@functools.partial(
    pl.kernel,
    out_type=jax.ShapeDtypeStruct((_NC, _PROWS, _D), jnp.float32),
    mesh=_mesh,
    scratch_types=[
        pltpu.VMEM((_CHUNK,), jnp.int32),
        pltpu.VMEM((_CHUNK,), jnp.int32),
        pltpu.VMEM((_CHUNK, _D), jnp.float32),
        pltpu.VMEM((_ZROWS, _D), jnp.float32),
        pltpu.VMEM_SHARED((_PROWS, _D), jnp.float32),
        pltpu.SemaphoreType.DMA,
    ],
)
def _propagate(src_hbm, dst_hbm, f_hbm, p_hbm, sidx_v, didx_v, rows_v, zbuf_v,
               part_sh, sem):
    c = lax.axis_index("c")
    s = lax.axis_index("s")
    wid = c * _NS + s

    @pl.loop(0, _ZROWS)
    def _zrow(i):
        for j in range(_D // 16):
            zbuf_v[i, pl.ds(j * 16, 16)] = jnp.zeros((16,), jnp.float32)

    @pl.loop(0, _ROWS_PER_TILE // _ZROWS)
    def _zpart(i):
        pltpu.sync_copy(
            zbuf_v, part_sh.at[pl.ds(s * _ROWS_PER_TILE + i * _ZROWS, _ZROWS)])

    plsc.subcore_barrier()

    base = wid * _EPT

    @pl.loop(0, _CHUNKS_PER_TILE)
    def _chunk(i):
        off = pl.multiple_of(base + i * _CHUNK, 8)
        pltpu.sync_copy(src_hbm.at[pl.ds(off, _CHUNK)], sidx_v)
        pltpu.sync_copy(dst_hbm.at[pl.ds(off, _CHUNK)], didx_v)
        pltpu.async_copy(f_hbm.at[sidx_v], rows_v, sem).wait()
        pltpu.sync_copy(rows_v, part_sh.at[didx_v], add=True)

    plsc.subcore_barrier()

    r = s * _ROWS_PER_TILE
    pltpu.sync_copy(part_sh.at[pl.ds(r, _ROWS_PER_TILE)],
                    p_hbm.at[c, pl.ds(r, _ROWS_PER_TILE)])


_BR = 2000


def _combine_body(p_ref, h_ref, g_ref, f_out, h_out):
    fnew = p_ref[0] + p_ref[1]
    f_out[...] = fnew
    h_out[...] = (h_ref[...] + (1.0 - _ALPHA) * fnew + _ALPHA * g_ref[...]) * (1.0 / _K)


_combine = pl.pallas_call(
    _combine_body,
    grid=(_N // _BR,),
    in_specs=[
        pl.BlockSpec((_NC, _BR, _D), lambda i: (0, i, 0)),
        pl.BlockSpec((_BR, _D), lambda i: (i, 0)),
        pl.BlockSpec((_BR, _D), lambda i: (i, 0)),
    ],
    out_specs=[
        pl.BlockSpec((_BR, _D), lambda i: (i, 0)),
        pl.BlockSpec((_BR, _D), lambda i: (i, 0)),
    ],
    out_shape=[
        jax.ShapeDtypeStruct((_N, _D), jnp.float32),
        jax.ShapeDtypeStruct((_N, _D), jnp.float32),
    ],
)


def _linear_body(h_ref, w_ref, b_ref, o_ref):
    o_ref[...] = lax.dot_general(
        h_ref[...], w_ref[...], (((1,), (1,)), ((), ())),
        preferred_element_type=jnp.float32) + b_ref[...]


_linear = pl.pallas_call(
    _linear_body,
    out_shape=jax.ShapeDtypeStruct((_N, _D), jnp.float32),
)


def kernel(feat, edge_index, W, b):
    dst = edge_index[0]
    src = edge_index[1]
    npad = _EPAD - _E
    src_p = jnp.concatenate([src, jnp.zeros((npad,), jnp.int32)])
    dst_p = jnp.concatenate([dst, jnp.full((npad,), _SCRATCH_ROW, jnp.int32)])

    f = feat
    h = jnp.zeros_like(feat)
    for _ in range(_K):
        p = _propagate(src_p, dst_p, f)
        f, h = _combine(p, h, feat)
    return _linear(h, W.astype(jnp.float32), b.reshape(1, _D))
